# baseline (speedup 1.0000x reference)
import functools

import jax
import jax.numpy as jnp
from jax import lax
from jax.experimental import pallas as pl
from jax.experimental.pallas import tpu as pltpu
from jax.experimental.pallas import tpu_sc as plsc

N_NODES = 10000
N_EDGES = 320000
D_FEAT = 128
D_EDGE = 16

NC, NS = 2, 16
NW = NC * NS
NG = N_EDGES // 128
NGW = 80
CHUNKG = 8
CHUNK = CHUNKG * 128
NCHUNK = NGW // CHUNKG
GWORDS = D_EDGE * 128
EPW = NGW * 128

_EBLK = 16000
_NBLK = N_EDGES // _EBLK
_GBLK = _EBLK // 128


def _node_proj_body(x8_ref, ws_ref, wd_ref, bt_ref, ei_ref,
                    psrc_ref, pdst_ref, xc_ref, ec_ref):
    x8 = x8_ref[...]
    psrc_ref[...] = (
        jnp.dot(x8, ws_ref[...], preferred_element_type=jnp.float32)
        + bt_ref[...]
    )
    pdst_ref[...] = jnp.dot(x8, wd_ref[...], preferred_element_type=jnp.float32)
    xc_ref[...] = x8
    ec_ref[...] = ei_ref[...]


def _node_proj(x8, ws_blk, wd_blk, b_tile, edge_index):
    return pl.pallas_call(
        _node_proj_body,
        out_shape=(
            jax.ShapeDtypeStruct((N_NODES // 8, 128), jnp.float32),
            jax.ShapeDtypeStruct((N_NODES // 8, 128), jnp.float32),
            jax.ShapeDtypeStruct((N_NODES // 8, 8 * D_FEAT), jnp.float32),
            jax.ShapeDtypeStruct((2, N_EDGES), jnp.int32),
        ),
    )(x8, ws_blk, wd_blk, b_tile, edge_index)


def _combine_body(xt_ref, g_ref, wt_ref, o_ref):
    xw = jnp.dot(wt_ref[...], xt_ref[...], preferred_element_type=jnp.float32)
    for j in range(_GBLK):
        sl = pl.ds(j * 128, 128)
        o_ref[:, sl] = xw[:, j * 128:(j + 1) * 128] + g_ref[j]


def _combine(x_t, g3, w_t):
    return pl.pallas_call(
        _combine_body,
        grid=(_NBLK,),
        in_specs=[
            pl.BlockSpec((D_EDGE, _EBLK), lambda i: (0, i)),
            pl.BlockSpec((_GBLK, D_EDGE, 128), lambda i: (i, 0, 0)),
            pl.BlockSpec((D_EDGE, D_EDGE), lambda i: (0, 0)),
        ],
        out_specs=pl.BlockSpec((D_EDGE, _EBLK), lambda i: (0, i)),
        out_shape=jax.ShapeDtypeStruct((D_EDGE, N_EDGES), jnp.float32),
    )(x_t, g3, w_t)


def _sc_body(psrc, pdst, e0, e1, out,
             idx0, idx1, s_a, s_b, d_a, d_b, t_a, t_b,
             gs_a, gs_b, os_a, os_b):
    cid = lax.axis_index("c")
    sid = lax.axis_index("s")
    wid = sid * NC + cid
    g_start = jnp.minimum(NG * wid // NW, NG - NGW)
    wbase = g_start * 128
    iotab = lax.iota(jnp.int32, 16) * 128

    pltpu.sync_copy(e0.at[pl.ds(wbase, EPW)], idx0)
    pltpu.sync_copy(e1.at[pl.ds(wbase, EPW)], idx1)

    S = (s_a, s_b)
    D = (d_a, d_b)
    T = (t_a, t_b)
    GS = (gs_a, gs_b)
    OS = (os_a, os_b)

    def start_gathers(i):
        p = i % 2
        cs = pltpu.async_copy(
            psrc.at[idx0.at[pl.ds(i * CHUNK, CHUNK)]], S[p], GS[p])
        cd = pltpu.async_copy(
            pdst.at[idx1.at[pl.ds(i * CHUNK, CHUNK)]], D[p], GS[p])
        return cs, cd

    pending = {0: start_gathers(0)}
    out_cp = {}
    for i in range(NCHUNK):
        p = i % 2
        if i + 1 < NCHUNK:
            pending[i + 1] = start_gathers(i + 1)
        cs, cd = pending.pop(i)
        cs.wait()
        cd.wait()
        if i >= 2:
            out_cp.pop(i - 2).wait()

        s_v, d_v, t_v = S[p], D[p], T[p]

        def body8(r8, c):
            g = r8 // 16
            base_s = g * GWORDS + (r8 % 16) * 8
            for k in range(8):
                e = r8 * 8 + k
                v = s_v[e, :] + d_v[e, :]
                plsc.store_scatter(t_v, [iotab + (base_s + k)], v)
            return c

        lax.fori_loop(0, CHUNK // 8, body8, 0)

        out_cp[i] = pltpu.async_copy(
            t_v,
            out.at[pl.ds((g_start + i * CHUNKG) * GWORDS, CHUNKG * GWORDS)],
            OS[p])

    out_cp.pop(NCHUNK - 2).wait()
    out_cp.pop(NCHUNK - 1).wait()


@functools.partial(
    pl.kernel,
    out_type=jax.ShapeDtypeStruct((N_EDGES * D_EDGE,), jnp.float32),
    mesh=plsc.VectorSubcoreMesh(core_axis_name="c", subcore_axis_name="s"),
    compiler_params=pltpu.CompilerParams(
        use_tc_tiling_on_sc=False, needs_layout_passes=False),
    scratch_types=[
        pltpu.VMEM((EPW,), jnp.int32),
        pltpu.VMEM((EPW,), jnp.int32),
        pltpu.VMEM((CHUNK, D_EDGE), jnp.float32),
        pltpu.VMEM((CHUNK, D_EDGE), jnp.float32),
        pltpu.VMEM((CHUNK, D_EDGE), jnp.float32),
        pltpu.VMEM((CHUNK, D_EDGE), jnp.float32),
        pltpu.VMEM((CHUNKG * GWORDS,), jnp.float32),
        pltpu.VMEM((CHUNKG * GWORDS,), jnp.float32),
        pltpu.SemaphoreType.DMA,
        pltpu.SemaphoreType.DMA,
        pltpu.SemaphoreType.DMA,
        pltpu.SemaphoreType.DMA,
    ],
)
def _sc_gather_sum(psrc, pdst, e0, e1, out,
                   idx0, idx1, s_a, s_b, d_a, d_b, t_a, t_b,
                   gs_a, gs_b, os_a, os_b):
    _sc_body(psrc, pdst, e0, e1, out,
             idx0, idx1, s_a, s_b, d_a, d_b, t_a, t_b,
             gs_a, gs_b, os_a, os_b)


def kernel(x_node, x_edge, edge_index, W, b):
    eye8 = jnp.eye(8, dtype=W.dtype)
    ws_blk = jnp.kron(eye8, W[:D_FEAT])
    wd_blk = jnp.kron(eye8, W[D_FEAT:2 * D_FEAT])
    b_tile = jnp.tile(b, 8)[None, :]
    w_t = W[2 * D_FEAT:].T

    x8 = x_node.reshape(N_NODES // 8, 8 * D_FEAT)
    psrc128, pdst128, xc8, edge_index_out = _node_proj(
        x8, ws_blk, wd_blk, b_tile, edge_index)

    e0 = edge_index[0]
    e1 = edge_index[1]
    gflat = _sc_gather_sum(
        psrc128.reshape(N_NODES, D_EDGE),
        pdst128.reshape(N_NODES, D_EDGE),
        e0, e1)
    g3 = gflat.reshape(NG, D_EDGE, 128)
    out_t = _combine(x_edge.T, g3, w_t)
    return (out_t.T, xc8.reshape(N_NODES, D_FEAT), edge_index_out)


# speedup vs baseline: 1.3957x; 1.3957x over previous
import functools

import jax
import jax.numpy as jnp
from jax import lax
from jax.experimental import pallas as pl
from jax.experimental.pallas import tpu as pltpu
from jax.experimental.pallas import tpu_sc as plsc

N_NODES = 10000
N_EDGES = 320000
D_FEAT = 128
D_EDGE = 16

NC, NS = 2, 16
NW = NC * NS
NG = N_EDGES // 128
NGW = 80
CHUNKG = 8
CHUNK = CHUNKG * 128
NCHUNK = NGW // CHUNKG
GWORDS = D_EDGE * 128
EPW = NGW * 128

_EBLK = 16000
_NBLK = N_EDGES // _EBLK
_GBLK = _EBLK // 128


def _node_proj_body(x8_ref, ws_ref, wd_ref, bt_ref, ei_ref,
                    psrc_ref, pdst_ref, xc_ref, ec_ref):
    x8 = x8_ref[...]
    psrc_ref[...] = (
        jnp.dot(x8, ws_ref[...], preferred_element_type=jnp.float32)
        + bt_ref[...]
    )
    pdst_ref[...] = jnp.dot(x8, wd_ref[...], preferred_element_type=jnp.float32)
    xc_ref[...] = x8
    ec_ref[...] = ei_ref[...]


def _node_proj(x8, ws_blk, wd_blk, b_tile, edge_index):
    return pl.pallas_call(
        _node_proj_body,
        out_shape=(
            jax.ShapeDtypeStruct((N_NODES // 8, 128), jnp.float32),
            jax.ShapeDtypeStruct((N_NODES // 8, 128), jnp.float32),
            jax.ShapeDtypeStruct((N_NODES // 8, 8 * D_FEAT), jnp.float32),
            jax.ShapeDtypeStruct((2, N_EDGES), jnp.int32),
        ),
    )(x8, ws_blk, wd_blk, b_tile, edge_index)


def _combine_body(xt_ref, g_ref, wt_ref, o_ref):
    xw = jnp.dot(wt_ref[...], xt_ref[...], preferred_element_type=jnp.float32)
    for j in range(_GBLK):
        sl = pl.ds(j * 128, 128)
        o_ref[:, sl] = xw[:, j * 128:(j + 1) * 128] + g_ref[j]


def _combine(x_t, g3, w_t):
    return pl.pallas_call(
        _combine_body,
        grid=(_NBLK,),
        in_specs=[
            pl.BlockSpec((D_EDGE, _EBLK), lambda i: (0, i)),
            pl.BlockSpec((_GBLK, D_EDGE, 128), lambda i: (i, 0, 0)),
            pl.BlockSpec((D_EDGE, D_EDGE), lambda i: (0, 0)),
        ],
        out_specs=pl.BlockSpec((D_EDGE, _EBLK), lambda i: (0, i)),
        out_shape=jax.ShapeDtypeStruct((D_EDGE, N_EDGES), jnp.float32),
    )(x_t, g3, w_t)


def _sc_body(psrc, pdst, e0, e1, out,
             idx0, idx1, s_a, s_b, d_a, d_b, t_a, t_b,
             gs_a, gs_b, os_a, os_b):
    cid = lax.axis_index("c")
    sid = lax.axis_index("s")
    wid = sid * NC + cid
    g_start = jnp.minimum(NG * wid // NW, NG - NGW)
    wbase = g_start * 128
    iota16 = lax.iota(jnp.int32, 16)

    pltpu.sync_copy(e0.at[pl.ds(wbase, EPW)], idx0)
    pltpu.sync_copy(e1.at[pl.ds(wbase, EPW)], idx1)

    S = (s_a, s_b)
    D = (d_a, d_b)
    T = (t_a, t_b)
    GS = (gs_a, gs_b)
    OS = (os_a, os_b)

    def start_gathers(i):
        p = i % 2
        cs = pltpu.async_copy(
            psrc.at[idx0.at[pl.ds(i * CHUNK, CHUNK)]], S[p], GS[p])
        cd = pltpu.async_copy(
            pdst.at[idx1.at[pl.ds(i * CHUNK, CHUNK)]], D[p], GS[p])
        return cs, cd

    pending = {0: start_gathers(0)}
    out_cp = {}
    for i in range(NCHUNK):
        p = i % 2
        if i + 1 < NCHUNK:
            pending[i + 1] = start_gathers(i + 1)
        cs, cd = pending.pop(i)
        cs.wait()
        cd.wait()
        if i >= 2:
            out_cp.pop(i - 2).wait()

        s_v, d_v, t_v = S[p], D[p], T[p]

        def body8(r8, c):
            row_vec = iota16 + (r8 // 16) * D_EDGE
            p_vec = jnp.full((16,), (r8 % 16) * 8, jnp.int32)
            for k in range(8):
                e = r8 * 8 + k
                v = s_v[e, :] + d_v[e, :]
                plsc.store_scatter(t_v, [row_vec, p_vec + k], v)
            return c

        lax.fori_loop(0, CHUNK // 8, body8, 0)

        out_cp[i] = pltpu.async_copy(
            t_v.at[:, pl.ds(0, 128)],
            out.at[pl.ds((g_start + i * CHUNKG) * D_EDGE, CHUNKG * D_EDGE)],
            OS[p])

    out_cp.pop(NCHUNK - 2).wait()
    out_cp.pop(NCHUNK - 1).wait()


@functools.partial(
    pl.kernel,
    out_type=jax.ShapeDtypeStruct((NG * D_EDGE, 128), jnp.float32),
    mesh=plsc.VectorSubcoreMesh(core_axis_name="c", subcore_axis_name="s"),
    compiler_params=pltpu.CompilerParams(
        use_tc_tiling_on_sc=False, needs_layout_passes=False),
    scratch_types=[
        pltpu.VMEM((EPW,), jnp.int32),
        pltpu.VMEM((EPW,), jnp.int32),
        pltpu.VMEM((CHUNK, D_EDGE), jnp.float32),
        pltpu.VMEM((CHUNK, D_EDGE), jnp.float32),
        pltpu.VMEM((CHUNK, D_EDGE), jnp.float32),
        pltpu.VMEM((CHUNK, D_EDGE), jnp.float32),
        pltpu.VMEM((CHUNKG * D_EDGE, 129), jnp.float32),
        pltpu.VMEM((CHUNKG * D_EDGE, 129), jnp.float32),
        pltpu.SemaphoreType.DMA,
        pltpu.SemaphoreType.DMA,
        pltpu.SemaphoreType.DMA,
        pltpu.SemaphoreType.DMA,
    ],
)
def _sc_gather_sum(psrc, pdst, e0, e1, out,
                   idx0, idx1, s_a, s_b, d_a, d_b, t_a, t_b,
                   gs_a, gs_b, os_a, os_b):
    _sc_body(psrc, pdst, e0, e1, out,
             idx0, idx1, s_a, s_b, d_a, d_b, t_a, t_b,
             gs_a, gs_b, os_a, os_b)


def kernel(x_node, x_edge, edge_index, W, b):
    eye8 = jnp.eye(8, dtype=W.dtype)
    ws_blk = jnp.kron(eye8, W[:D_FEAT])
    wd_blk = jnp.kron(eye8, W[D_FEAT:2 * D_FEAT])
    b_tile = jnp.tile(b, 8)[None, :]
    w_t = W[2 * D_FEAT:].T

    x8 = x_node.reshape(N_NODES // 8, 8 * D_FEAT)
    psrc128, pdst128, xc8, edge_index_out = _node_proj(
        x8, ws_blk, wd_blk, b_tile, edge_index)

    e0 = edge_index[0]
    e1 = edge_index[1]
    g2 = _sc_gather_sum(
        psrc128.reshape(N_NODES, D_EDGE),
        pdst128.reshape(N_NODES, D_EDGE),
        e0, e1)
    g3 = g2.reshape(NG, D_EDGE, 128)
    out_t = _combine(x_edge.T, g3, w_t)
    return (out_t.T, xc8.reshape(N_NODES, D_FEAT), edge_index_out)


# speedup vs baseline: 1.7528x; 1.2558x over previous
import functools

import jax
import jax.numpy as jnp
from jax import lax
from jax.experimental import pallas as pl
from jax.experimental.pallas import tpu as pltpu
from jax.experimental.pallas import tpu_sc as plsc

N_NODES = 10000
N_EDGES = 320000
D_FEAT = 128
D_EDGE = 16

NC, NS = 2, 16
NW = NC * NS
NG = N_EDGES // 128
NGW = 80
CHUNKG = 8
CHUNK = CHUNKG * 128
NCHUNK = NGW // CHUNKG
GWORDS = D_EDGE * 128
EPW = NGW * 128

_EBLK = 16000
_NBLK = N_EDGES // _EBLK
_GBLK = _EBLK // 128


def _node_proj_body(x8_ref, ws_ref, wd_ref, bt_ref, ei_ref,
                    psrc_ref, pdst_ref, xc_ref, ec_ref):
    x8 = x8_ref[...]
    psrc_ref[...] = (
        jnp.dot(x8, ws_ref[...], preferred_element_type=jnp.float32)
        + bt_ref[...]
    )
    pdst_ref[...] = jnp.dot(x8, wd_ref[...], preferred_element_type=jnp.float32)
    xc_ref[...] = x8
    ec_ref[...] = ei_ref[...]


def _node_proj(x8, ws_blk, wd_blk, b_tile, edge_index):
    return pl.pallas_call(
        _node_proj_body,
        out_shape=(
            jax.ShapeDtypeStruct((N_NODES // 8, 128), jnp.float32),
            jax.ShapeDtypeStruct((N_NODES // 8, 128), jnp.float32),
            jax.ShapeDtypeStruct((N_NODES // 8, 8 * D_FEAT), jnp.float32),
            jax.ShapeDtypeStruct((2, N_EDGES), jnp.int32),
        ),
    )(x8, ws_blk, wd_blk, b_tile, edge_index)


def _combine_body(xt_ref, g_ref, wt_ref, o_ref):
    xw = jnp.dot(wt_ref[...], xt_ref[...], preferred_element_type=jnp.float32)
    for j in range(_GBLK):
        sl = pl.ds(j * 128, 128)
        o_ref[:, sl] = xw[:, j * 128:(j + 1) * 128] + g_ref[j]


def _combine(x_t, g3, w_t):
    return pl.pallas_call(
        _combine_body,
        grid=(_NBLK,),
        in_specs=[
            pl.BlockSpec((D_EDGE, _EBLK), lambda i: (0, i)),
            pl.BlockSpec((_GBLK, D_EDGE, 128), lambda i: (i, 0, 0)),
            pl.BlockSpec((D_EDGE, D_EDGE), lambda i: (0, 0)),
        ],
        out_specs=pl.BlockSpec((D_EDGE, _EBLK), lambda i: (0, i)),
        out_shape=jax.ShapeDtypeStruct((D_EDGE, N_EDGES), jnp.float32),
    )(x_t, g3, w_t)


def _sc_body(psrc, pdst, e0, e1, out,
             idx0, idx1, s_a, s_b, d_a, d_b, t_a, t_b,
             gs_a, gs_b, os_a, os_b):
    cid = lax.axis_index("c")
    sid = lax.axis_index("s")
    wid = sid * NC + cid
    g_start = jnp.minimum(NG * wid // NW, NG - NGW)
    wbase = g_start * 128
    iota16 = lax.iota(jnp.int32, 16)

    pltpu.sync_copy(e0.at[pl.ds(wbase, EPW)], idx0)
    pltpu.sync_copy(e1.at[pl.ds(wbase, EPW)], idx1)

    S = (s_a, s_b)
    D = (d_a, d_b)
    T = (t_a, t_b)
    GS = (gs_a, gs_b)
    OS = (os_a, os_b)

    def start_gathers(i):
        p = i % 2
        cs = pltpu.async_copy(
            psrc.at[idx0.at[pl.ds(i * CHUNK, CHUNK)]], S[p], GS[p])
        cd = pltpu.async_copy(
            pdst.at[idx1.at[pl.ds(i * CHUNK, CHUNK)]], D[p], GS[p])
        return cs, cd

    pending = {0: start_gathers(0)}
    out_cp = {}
    for i in range(NCHUNK):
        p = i % 2
        if i + 1 < NCHUNK:
            pending[i + 1] = start_gathers(i + 1)
        cs, cd = pending.pop(i)
        cs.wait()
        cd.wait()
        if i >= 2:
            out_cp.pop(i - 2).wait()

        s_v, d_v, t_v = S[p], D[p], T[p]

        @plsc.parallel_loop(0, CHUNK // 8, unroll=2)
        def _(r8):
            row_vec = iota16 + (r8 // 16) * D_EDGE
            p_vec = jnp.full((16,), (r8 % 16) * 8, jnp.int32)
            for k in range(8):
                e = r8 * 8 + k
                v = s_v[e, :] + d_v[e, :]
                plsc.store_scatter(t_v, [row_vec, p_vec + k], v)

        out_cp[i] = pltpu.async_copy(
            t_v.at[:, pl.ds(0, 128)],
            out.at[pl.ds((g_start + i * CHUNKG) * D_EDGE, CHUNKG * D_EDGE)],
            OS[p])

    out_cp.pop(NCHUNK - 2).wait()
    out_cp.pop(NCHUNK - 1).wait()


@functools.partial(
    pl.kernel,
    out_type=jax.ShapeDtypeStruct((NG * D_EDGE, 128), jnp.float32),
    mesh=plsc.VectorSubcoreMesh(core_axis_name="c", subcore_axis_name="s"),
    compiler_params=pltpu.CompilerParams(
        use_tc_tiling_on_sc=False, needs_layout_passes=False),
    scratch_types=[
        pltpu.VMEM((EPW,), jnp.int32),
        pltpu.VMEM((EPW,), jnp.int32),
        pltpu.VMEM((CHUNK, D_EDGE), jnp.float32),
        pltpu.VMEM((CHUNK, D_EDGE), jnp.float32),
        pltpu.VMEM((CHUNK, D_EDGE), jnp.float32),
        pltpu.VMEM((CHUNK, D_EDGE), jnp.float32),
        pltpu.VMEM((CHUNKG * D_EDGE, 129), jnp.float32),
        pltpu.VMEM((CHUNKG * D_EDGE, 129), jnp.float32),
        pltpu.SemaphoreType.DMA,
        pltpu.SemaphoreType.DMA,
        pltpu.SemaphoreType.DMA,
        pltpu.SemaphoreType.DMA,
    ],
)
def _sc_gather_sum(psrc, pdst, e0, e1, out,
                   idx0, idx1, s_a, s_b, d_a, d_b, t_a, t_b,
                   gs_a, gs_b, os_a, os_b):
    _sc_body(psrc, pdst, e0, e1, out,
             idx0, idx1, s_a, s_b, d_a, d_b, t_a, t_b,
             gs_a, gs_b, os_a, os_b)


def kernel(x_node, x_edge, edge_index, W, b):
    eye8 = jnp.eye(8, dtype=W.dtype)
    ws_blk = jnp.kron(eye8, W[:D_FEAT])
    wd_blk = jnp.kron(eye8, W[D_FEAT:2 * D_FEAT])
    b_tile = jnp.tile(b, 8)[None, :]
    w_t = W[2 * D_FEAT:].T

    x8 = x_node.reshape(N_NODES // 8, 8 * D_FEAT)
    psrc128, pdst128, xc8, edge_index_out = _node_proj(
        x8, ws_blk, wd_blk, b_tile, edge_index)

    e0 = edge_index[0]
    e1 = edge_index[1]
    g2 = _sc_gather_sum(
        psrc128.reshape(N_NODES, D_EDGE),
        pdst128.reshape(N_NODES, D_EDGE),
        e0, e1)
    g3 = g2.reshape(NG, D_EDGE, 128)
    out_t = _combine(x_edge.T, g3, w_t)
    return (out_t.T, xc8.reshape(N_NODES, D_FEAT), edge_index_out)


# speedup vs baseline: 1.7610x; 1.0047x over previous
import functools

import jax
import jax.numpy as jnp
from jax import lax
from jax.experimental import pallas as pl
from jax.experimental.pallas import tpu as pltpu
from jax.experimental.pallas import tpu_sc as plsc

N_NODES = 10000
N_EDGES = 320000
D_FEAT = 128
D_EDGE = 16

NC, NS = 2, 16
NW = NC * NS
NG = N_EDGES // 128
NGW = 80
CHUNKG = 8
CHUNK = CHUNKG * 128
NCHUNK = NGW // CHUNKG
GWORDS = D_EDGE * 128
EPW = NGW * 128

_EBLK = 16000
_NBLK = N_EDGES // _EBLK
_GBLK = _EBLK // 128


def _node_proj_body(x8_ref, ws_ref, wd_ref, bt_ref, ei_ref,
                    psrc_ref, pdst_ref, xc_ref, ec_ref):
    x8 = x8_ref[...]
    psrc_ref[...] = (
        jnp.dot(x8, ws_ref[...], preferred_element_type=jnp.float32)
        + bt_ref[...]
    )
    pdst_ref[...] = jnp.dot(x8, wd_ref[...], preferred_element_type=jnp.float32)
    xc_ref[...] = x8
    ec_ref[...] = ei_ref[...]


def _node_proj(x8, ws_blk, wd_blk, b_tile, edge_index):
    return pl.pallas_call(
        _node_proj_body,
        out_shape=(
            jax.ShapeDtypeStruct((N_NODES // 8, 128), jnp.float32),
            jax.ShapeDtypeStruct((N_NODES // 8, 128), jnp.float32),
            jax.ShapeDtypeStruct((N_NODES // 8, 8 * D_FEAT), jnp.float32),
            jax.ShapeDtypeStruct((2, N_EDGES), jnp.int32),
        ),
    )(x8, ws_blk, wd_blk, b_tile, edge_index)


def _combine_body(xt_ref, g_ref, wt_ref, o_ref):
    xw = jnp.dot(wt_ref[...], xt_ref[...], preferred_element_type=jnp.float32)
    for j in range(_GBLK):
        sl = pl.ds(j * 128, 128)
        o_ref[:, sl] = xw[:, j * 128:(j + 1) * 128] + g_ref[j]


def _combine(x_t, g3, w_t):
    return pl.pallas_call(
        _combine_body,
        grid=(_NBLK,),
        in_specs=[
            pl.BlockSpec((D_EDGE, _EBLK), lambda i: (0, i)),
            pl.BlockSpec((_GBLK, D_EDGE, 128), lambda i: (i, 0, 0)),
            pl.BlockSpec((D_EDGE, D_EDGE), lambda i: (0, 0)),
        ],
        out_specs=pl.BlockSpec((D_EDGE, _EBLK), lambda i: (0, i)),
        out_shape=jax.ShapeDtypeStruct((D_EDGE, N_EDGES), jnp.float32),
    )(x_t, g3, w_t)


def _sc_body(psrc, pdst, ei2, out,
             idxv, s_a, s_b, d_a, d_b, t_a, t_b,
             gs_a, gs_b, os_a, os_b):
    cid = lax.axis_index("c")
    sid = lax.axis_index("s")
    wid = sid * NC + cid
    g_start = jnp.minimum(NG * wid // NW, NG - NGW)
    wbase = g_start * 128
    iota16 = lax.iota(jnp.int32, 16)

    pltpu.sync_copy(ei2.at[pl.ds(g_start, NGW)], idxv)

    S = (s_a, s_b)
    D = (d_a, d_b)
    T = (t_a, t_b)
    GS = (gs_a, gs_b)
    OS = (os_a, os_b)

    def start_gathers(i):
        p = i % 2
        cps = []
        for g in range(CHUNKG):
            row = i * CHUNKG + g
            cps.append(pltpu.async_copy(
                psrc.at[idxv.at[row, pl.ds(0, 128)]],
                S[p].at[pl.ds(g * 128, 128)], GS[p]))
            cps.append(pltpu.async_copy(
                pdst.at[idxv.at[row, pl.ds(128, 128)]],
                D[p].at[pl.ds(g * 128, 128)], GS[p]))
        return cps

    pending = {0: start_gathers(0)}
    out_cp = {}
    for i in range(NCHUNK):
        p = i % 2
        if i + 1 < NCHUNK:
            pending[i + 1] = start_gathers(i + 1)
        for cp in pending.pop(i):
            cp.wait()
        if i >= 2:
            out_cp.pop(i - 2).wait()

        s_v, d_v, t_v = S[p], D[p], T[p]

        @plsc.parallel_loop(0, CHUNK // 8, unroll=2)
        def _(r8):
            row_vec = iota16 + (r8 // 16) * D_EDGE
            p_vec = jnp.full((16,), (r8 % 16) * 8, jnp.int32)
            for k in range(8):
                e = r8 * 8 + k
                v = s_v[e, :] + d_v[e, :]
                plsc.store_scatter(t_v, [row_vec, p_vec + k], v)

        out_cp[i] = pltpu.async_copy(
            t_v.at[:, pl.ds(0, 128)],
            out.at[pl.ds((g_start + i * CHUNKG) * D_EDGE, CHUNKG * D_EDGE)],
            OS[p])

    out_cp.pop(NCHUNK - 2).wait()
    out_cp.pop(NCHUNK - 1).wait()


@functools.partial(
    pl.kernel,
    out_type=jax.ShapeDtypeStruct((NG * D_EDGE, 128), jnp.float32),
    mesh=plsc.VectorSubcoreMesh(core_axis_name="c", subcore_axis_name="s"),
    compiler_params=pltpu.CompilerParams(
        use_tc_tiling_on_sc=False, needs_layout_passes=False),
    scratch_types=[
        pltpu.VMEM((NGW, 256), jnp.int32),
        pltpu.VMEM((CHUNK, D_EDGE), jnp.float32),
        pltpu.VMEM((CHUNK, D_EDGE), jnp.float32),
        pltpu.VMEM((CHUNK, D_EDGE), jnp.float32),
        pltpu.VMEM((CHUNK, D_EDGE), jnp.float32),
        pltpu.VMEM((CHUNKG * D_EDGE, 129), jnp.float32),
        pltpu.VMEM((CHUNKG * D_EDGE, 129), jnp.float32),
        pltpu.SemaphoreType.DMA,
        pltpu.SemaphoreType.DMA,
        pltpu.SemaphoreType.DMA,
        pltpu.SemaphoreType.DMA,
    ],
)
def _sc_gather_sum(psrc, pdst, ei2, out,
                   idxv, s_a, s_b, d_a, d_b, t_a, t_b,
                   gs_a, gs_b, os_a, os_b):
    _sc_body(psrc, pdst, ei2, out,
             idxv, s_a, s_b, d_a, d_b, t_a, t_b,
             gs_a, gs_b, os_a, os_b)


def kernel(x_node, x_edge, edge_index, W, b):
    eye8 = jnp.eye(8, dtype=W.dtype)
    ws_blk = jnp.kron(eye8, W[:D_FEAT])
    wd_blk = jnp.kron(eye8, W[D_FEAT:2 * D_FEAT])
    b_tile = jnp.tile(b, 8)[None, :]
    w_t = W[2 * D_FEAT:].T

    x8 = x_node.reshape(N_NODES // 8, 8 * D_FEAT)
    psrc128, pdst128, xc8, edge_index_out = _node_proj(
        x8, ws_blk, wd_blk, b_tile, edge_index)

    ei2 = edge_index.reshape(2, NG, 128).transpose(1, 0, 2).reshape(NG, 256)
    g2 = _sc_gather_sum(
        psrc128.reshape(N_NODES, D_EDGE),
        pdst128.reshape(N_NODES, D_EDGE),
        ei2)
    g3 = g2.reshape(NG, D_EDGE, 128)
    out_t = _combine(x_edge.T, g3, w_t)
    return (out_t.T, xc8.reshape(N_NODES, D_FEAT), edge_index_out)
